# baseline (speedup 1.0000x reference)
import functools

import jax
import jax.numpy as jnp
from jax import lax
from jax.experimental import pallas as pl
from jax.experimental.pallas import tpu as pltpu
from jax.experimental.pallas import tpu_sc as plsc


def _sc_relayout(tableT, tail2):
    E, V = tableT.shape
    CH = 128
    n_full = V // CH
    tail = V - n_full * CH
    info = plsc.get_sparse_core_info()
    NC, NS = info.num_cores, info.num_subcores
    NW = NC * NS
    per = n_full // NW
    extra = n_full - per * NW
    mesh = plsc.VectorSubcoreMesh(core_axis_name="c", subcore_axis_name="s")

    @functools.partial(
        pl.kernel,
        mesh=mesh,
        out_type=jax.ShapeDtypeStruct((V // 2, 2 * E), jnp.float32),
        scratch_types=[
            pltpu.VMEM((2, E, CH), jnp.float32),
            pltpu.VMEM((2, CH // 2, 2 * E), jnp.float32),
            pltpu.SemaphoreType.DMA,
            pltpu.SemaphoreType.DMA,
        ],
        compiler_params=pltpu.CompilerParams(needs_layout_passes=False),
    )
    def relayout_kernel(tableT_hbm, tail2_hbm, out_hbm,
                        inbuf, outbuf, sem_i, sem_o):
        wid = lax.axis_index("s") * NC + lax.axis_index("c")
        base = wid * per + jnp.minimum(wid, extra)
        n_my = per + jnp.where(wid < extra, 1, 0)
        lanes = lax.broadcasted_iota(jnp.int32, (16,), 0)

        def col0_of(g):
            return pl.multiple_of(g * CH, CH)

        def row0_of(g):
            return pl.multiple_of(g * (CH // 2), CH // 2)

        def start_in(g, s):
            return pltpu.make_async_copy(
                tableT_hbm.at[:, pl.ds(col0_of(g), CH)],
                inbuf.at[s], sem_i)

        def transpose_block(s):
            for p in range(CH // 2):
                for h in range(2):
                    col = jnp.full((16,), 2 * p + h, jnp.int32)
                    for k in range(E // 16):
                        vals = plsc.load_gather(
                            inbuf.at[s], [k * 16 + lanes, col])
                        outbuf.at[s][p, pl.ds(h * E + k * 16, 16)] = vals

        def start_out(g, s):
            return pltpu.make_async_copy(
                outbuf.at[s],
                out_hbm.at[pl.ds(row0_of(g), CH // 2)], sem_o)

        n_tot = n_my

        def gid(c):
            return base + c

        start_in(gid(0), 0).start()

        def body(c, _):
            s = jax.lax.rem(c, 2)

            @pl.when(c + 1 < n_tot)
            def _prefetch():
                start_in(gid(c + 1), 1 - s).start()

            start_in(gid(c), s).wait()

            @pl.when(c >= 2)
            def _drain_out():
                start_out(gid(c - 2), s).wait()

            @pl.when(s == 0)
            def _t0():
                transpose_block(0)

            @pl.when(s == 1)
            def _t1():
                transpose_block(1)

            start_out(gid(c), s).start()
            return 0

        lax.fori_loop(0, n_tot, body, 0)
        start_out(0, 0).wait()
        start_out(0, 1).wait()

        if tail:
            @pl.when(wid == 5)
            def _tail_block():
                nt = tail // 2
                pltpu.async_copy(
                    tail2_hbm, outbuf.at[0, pl.ds(0, nt)], sem_i).wait()
                pltpu.async_copy(
                    outbuf.at[0, pl.ds(0, nt)],
                    out_hbm.at[pl.ds(V // 2 - nt, nt)], sem_o).wait()

    return relayout_kernel(tableT, tail2)


def _sc_gather(table, idx_pad, nchunks, n_rows):
    NW, KMAX, CH = idx_pad.shape
    _, D = table.shape
    info = plsc.get_sparse_core_info()
    NC = info.num_cores
    full = nchunks // NW
    extra = nchunks - full * NW
    mesh = plsc.VectorSubcoreMesh(core_axis_name="c", subcore_axis_name="s")

    @functools.partial(
        pl.kernel,
        mesh=mesh,
        out_type=jax.ShapeDtypeStruct((n_rows, D), table.dtype),
        scratch_types=[
            pltpu.VMEM((KMAX, CH), jnp.int32),
            pltpu.VMEM((2, CH, D), table.dtype),
            pltpu.VMEM((CH, D), table.dtype),
            pltpu.SemaphoreType.DMA,
            pltpu.SemaphoreType.DMA,
            pltpu.SemaphoreType.DMA,
        ],
    )
    def gather_kernel(table_hbm, idx_hbm, out_hbm,
                      idx_v, rows_v, rows_x, sem, semo, semx):
        wid = lax.axis_index("s") * NC + lax.axis_index("c")
        start = wid * full + jnp.minimum(wid, extra)
        pltpu.sync_copy(idx_hbm.at[wid], idx_v)

        def gather_in(j):
            return pltpu.make_async_copy(
                table_hbm.at[idx_v.at[j]], rows_v.at[j % 2], sem)

        def copy_out(j):
            return pltpu.make_async_copy(
                rows_v.at[j % 2],
                out_hbm.at[pl.ds((start + j) * CH, CH)], semo)

        if extra:
            @pl.when(wid < extra)
            def _extra_chunk():
                pltpu.async_copy(
                    table_hbm.at[idx_v.at[full]], rows_x, semx).wait()
                pltpu.sync_copy(
                    rows_x, out_hbm.at[pl.ds((start + full) * CH, CH)])

        gather_in(0).start()
        outs = []
        for j in range(full):
            gather_in(j).wait()
            if j >= 1:
                outs[j - 1].wait()
            if j + 1 < full:
                gather_in(j + 1).start()
            o = copy_out(j)
            o.start()
            outs.append(o)
        if full:
            outs[-1].wait()

    return gather_kernel(table, idx_pad)


def _lstm_body(U, E, xs_ref, m_ref, h0_ref, c0_ref, W_ref, Ur_ref, b_ref,
               out_ref, hf_ref, cf_ref, h_scr, c_scr):
    t = pl.program_id(0)
    S = pl.num_programs(0)

    @pl.when(t == 0)
    def _init():
        h_scr[...] = h0_ref[...]
        c_scr[...] = c0_ref[...]

    x2 = xs_ref[0]
    m = m_ref[0]
    xlo = x2[:, :E]
    x = xlo + m * (x2[:, E:] - xlo)
    h = h_scr[...]
    z = (jnp.dot(x, W_ref[...], preferred_element_type=jnp.float32)
         + jnp.dot(h, Ur_ref[...], preferred_element_type=jnp.float32)
         + b_ref[...])
    i = jax.nn.sigmoid(z[:, :U])
    f = jax.nn.sigmoid(z[:, U:2 * U])
    g = jnp.tanh(z[:, 2 * U:3 * U])
    o = jax.nn.sigmoid(z[:, 3 * U:])
    c_new = f * c_scr[...] + i * g
    h_new = o * jnp.tanh(c_new)
    h_scr[...] = h_new
    c_scr[...] = c_new
    out_ref[0] = h_new

    @pl.when(t == S - 1)
    def _fin():
        hf_ref[...] = h_new
        cf_ref[...] = c_new


def _lstm(xs2, msel, h0, c0, W, Ur, b2):
    S, B, E2 = xs2.shape
    E = E2 // 2
    U = h0.shape[1]
    G = 4 * U
    return pl.pallas_call(
        functools.partial(_lstm_body, U, E),
        grid=(S,),
        in_specs=[
            pl.BlockSpec((1, B, E2), lambda t: (t, 0, 0)),
            pl.BlockSpec((1, B, 1), lambda t: (t, 0, 0)),
            pl.BlockSpec((B, U), lambda t: (0, 0)),
            pl.BlockSpec((B, U), lambda t: (0, 0)),
            pl.BlockSpec((E, G), lambda t: (0, 0)),
            pl.BlockSpec((U, G), lambda t: (0, 0)),
            pl.BlockSpec((1, G), lambda t: (0, 0)),
        ],
        out_specs=[
            pl.BlockSpec((1, B, U), lambda t: (t, 0, 0)),
            pl.BlockSpec((B, U), lambda t: (0, 0)),
            pl.BlockSpec((B, U), lambda t: (0, 0)),
        ],
        out_shape=[
            jax.ShapeDtypeStruct((S, B, U), jnp.float32),
            jax.ShapeDtypeStruct((B, U), jnp.float32),
            jax.ShapeDtypeStruct((B, U), jnp.float32),
        ],
        scratch_shapes=[
            pltpu.VMEM((B, U), jnp.float32),
            pltpu.VMEM((B, U), jnp.float32),
        ],
        compiler_params=pltpu.CompilerParams(
            dimension_semantics=("arbitrary",)),
    )(xs2, msel, h0, c0, W, Ur, b2)


def kernel(sequence, state_h, state_c, emb_table, W, Ur, b):
    B, S = sequence.shape
    _, E = emb_table.shape
    U = state_h.shape[1]
    N = B * S
    info = plsc.get_sparse_core_info()
    NW = info.num_cores * info.num_subcores

    seqT = sequence.T.astype(jnp.int32)
    pair = (seqT >> 1).reshape(N // 128, 128)
    msel = (seqT & 1).astype(jnp.float32).reshape(S, B, 1)
    nchunks = pair.shape[0]
    full, extra = nchunks // NW, nchunks % NW
    kmax = full + (1 if extra else 0)
    starts = jnp.arange(NW) * full + jnp.minimum(jnp.arange(NW), extra)
    row_ids = jnp.minimum(starts[:, None] + jnp.arange(kmax)[None, :],
                          nchunks - 1)
    idx_pad = pair[row_ids]

    V = emb_table.shape[0]
    ntail = V - (V // 128) * 128
    tail2 = emb_table[V - ntail:].reshape(ntail // 2, 2 * E)
    table2 = _sc_relayout(emb_table.T, tail2)
    xs2 = _sc_gather(table2, idx_pad, nchunks, N).reshape(S, B, 2 * E)
    hs, hf, cf = _lstm(xs2, msel, state_h, state_c, W, Ur, b.reshape(1, -1))
    return (jnp.swapaxes(hs, 0, 1), hf, cf)


# speedup vs baseline: 2.2954x; 2.2954x over previous
import functools

import jax
import jax.numpy as jnp
from jax import lax
from jax.experimental import pallas as pl
from jax.experimental.pallas import tpu as pltpu
from jax.experimental.pallas import tpu_sc as plsc


def _sc_relayout(tableT, tail2):
    E, V = tableT.shape
    CH = 128
    n_full = V // CH
    tail = V - n_full * CH
    info = plsc.get_sparse_core_info()
    NC, NS = info.num_cores, info.num_subcores
    NW = NC * NS
    per = n_full // NW
    extra = n_full - per * NW
    mesh = plsc.VectorSubcoreMesh(core_axis_name="c", subcore_axis_name="s")

    @functools.partial(
        pl.kernel,
        mesh=mesh,
        out_type=jax.ShapeDtypeStruct((V // 2, 2 * E), jnp.float32),
        scratch_types=[
            pltpu.VMEM((2, E, CH), jnp.float32),
            pltpu.VMEM((2, CH // 2, 2 * E), jnp.float32),
            pltpu.SemaphoreType.DMA,
            pltpu.SemaphoreType.DMA,
        ],
        compiler_params=pltpu.CompilerParams(needs_layout_passes=False),
    )
    def relayout_kernel(tableT_hbm, tail2_hbm, out_hbm,
                        inbuf, outbuf, sem_i, sem_o):
        wid = lax.axis_index("s") * NC + lax.axis_index("c")
        base = wid * per + jnp.minimum(wid, extra)
        n_my = per + jnp.where(wid < extra, 1, 0)
        lanes = lax.broadcasted_iota(jnp.int32, (16,), 0)

        def col0_of(g):
            return pl.multiple_of(g * CH, CH)

        def row0_of(g):
            return pl.multiple_of(g * (CH // 2), CH // 2)

        def start_in(g, s):
            return pltpu.make_async_copy(
                tableT_hbm.at[:, pl.ds(col0_of(g), CH)],
                inbuf.at[s], sem_i)

        def transpose_block(s):
            for p in range(CH // 2):
                for h in range(2):
                    col = jnp.full((16,), 2 * p + h, jnp.int32)
                    for k in range(E // 16):
                        vals = plsc.load_gather(
                            inbuf.at[s], [k * 16 + lanes, col])
                        outbuf.at[s][p, pl.ds(h * E + k * 16, 16)] = vals

        def start_out(g, s):
            return pltpu.make_async_copy(
                outbuf.at[s],
                out_hbm.at[pl.ds(row0_of(g), CH // 2)], sem_o)

        n_tot = n_my

        def gid(c):
            return base + c

        start_in(gid(0), 0).start()

        def body(c, _):
            s = jax.lax.rem(c, 2)

            @pl.when(c + 1 < n_tot)
            def _prefetch():
                start_in(gid(c + 1), 1 - s).start()

            start_in(gid(c), s).wait()

            @pl.when(c >= 2)
            def _drain_out():
                start_out(gid(c - 2), s).wait()

            @pl.when(s == 0)
            def _t0():
                transpose_block(0)

            @pl.when(s == 1)
            def _t1():
                transpose_block(1)

            start_out(gid(c), s).start()
            return 0

        lax.fori_loop(0, n_tot, body, 0)
        start_out(0, 0).wait()
        start_out(0, 1).wait()

        if tail:
            @pl.when(wid == 5)
            def _tail_block():
                nt = tail // 2
                pltpu.async_copy(
                    tail2_hbm, outbuf.at[0, pl.ds(0, nt)], sem_i).wait()
                pltpu.async_copy(
                    outbuf.at[0, pl.ds(0, nt)],
                    out_hbm.at[pl.ds(V // 2 - nt, nt)], sem_o).wait()

    return relayout_kernel(tableT, tail2)


def _sc_gather(table, idx_pad, nchunks, n_rows):
    NW, KMAX, CH = idx_pad.shape
    _, D = table.shape
    info = plsc.get_sparse_core_info()
    NC = info.num_cores
    full = nchunks // NW
    extra = nchunks - full * NW
    mesh = plsc.VectorSubcoreMesh(core_axis_name="c", subcore_axis_name="s")

    @functools.partial(
        pl.kernel,
        mesh=mesh,
        out_type=jax.ShapeDtypeStruct((n_rows, D), table.dtype),
        scratch_types=[
            pltpu.VMEM((KMAX, CH), jnp.int32),
            pltpu.VMEM((2, CH, D), table.dtype),
            pltpu.VMEM((CH, D), table.dtype),
            pltpu.SemaphoreType.DMA,
            pltpu.SemaphoreType.DMA,
            pltpu.SemaphoreType.DMA,
        ],
    )
    def gather_kernel(table_hbm, idx_hbm, out_hbm,
                      idx_v, rows_v, rows_x, sem, semo, semx):
        wid = lax.axis_index("s") * NC + lax.axis_index("c")
        start = wid * full + jnp.minimum(wid, extra)
        pltpu.sync_copy(idx_hbm.at[wid], idx_v)

        def gather_in(j):
            return pltpu.make_async_copy(
                table_hbm.at[idx_v.at[j]], rows_v.at[j % 2], sem)

        def copy_out(j):
            return pltpu.make_async_copy(
                rows_v.at[j % 2],
                out_hbm.at[pl.ds((start + j) * CH, CH)], semo)

        if extra:
            @pl.when(wid < extra)
            def _extra_chunk():
                pltpu.async_copy(
                    table_hbm.at[idx_v.at[full]], rows_x, semx).wait()
                pltpu.sync_copy(
                    rows_x, out_hbm.at[pl.ds((start + full) * CH, CH)])

        gather_in(0).start()
        outs = []
        for j in range(full):
            gather_in(j).wait()
            if j >= 1:
                outs[j - 1].wait()
            if j + 1 < full:
                gather_in(j + 1).start()
            o = copy_out(j)
            o.start()
            outs.append(o)
        if full:
            outs[-1].wait()

    return gather_kernel(table, idx_pad)


def _lstm_body(U, E, xs_ref, m_ref, h0_ref, c0_ref, W_ref, Ur_ref, b_ref,
               out_ref, hf_ref, cf_ref, h_scr, c_scr):
    t = pl.program_id(0)
    S = pl.num_programs(0)

    @pl.when(t == 0)
    def _init():
        h_scr[...] = h0_ref[...]
        c_scr[...] = c0_ref[...]

    x2 = xs_ref[0]
    m = m_ref[0]
    xlo = x2[:, :E]
    x = xlo + m * (x2[:, E:] - xlo)
    h = h_scr[...]
    z = (jnp.dot(x, W_ref[...], preferred_element_type=jnp.float32)
         + jnp.dot(h, Ur_ref[...], preferred_element_type=jnp.float32)
         + b_ref[...])
    i = jax.nn.sigmoid(z[:, :U])
    f = jax.nn.sigmoid(z[:, U:2 * U])
    g = jnp.tanh(z[:, 2 * U:3 * U])
    o = jax.nn.sigmoid(z[:, 3 * U:])
    c_new = f * c_scr[...] + i * g
    h_new = o * jnp.tanh(c_new)
    h_scr[...] = h_new
    c_scr[...] = c_new
    out_ref[0] = h_new

    @pl.when(t == S - 1)
    def _fin():
        hf_ref[...] = h_new
        cf_ref[...] = c_new


def _lstm(xs2, msel, h0, c0, W, Ur, b2):
    S, B, E2 = xs2.shape
    E = E2 // 2
    U = h0.shape[1]
    G = 4 * U
    return pl.pallas_call(
        functools.partial(_lstm_body, U, E),
        grid=(S,),
        in_specs=[
            pl.BlockSpec((1, B, E2), lambda t: (t, 0, 0)),
            pl.BlockSpec((1, B, 1), lambda t: (t, 0, 0)),
            pl.BlockSpec((B, U), lambda t: (0, 0)),
            pl.BlockSpec((B, U), lambda t: (0, 0)),
            pl.BlockSpec((E, G), lambda t: (0, 0)),
            pl.BlockSpec((U, G), lambda t: (0, 0)),
            pl.BlockSpec((1, G), lambda t: (0, 0)),
        ],
        out_specs=[
            pl.BlockSpec((1, B, U), lambda t: (t, 0, 0)),
            pl.BlockSpec((B, U), lambda t: (0, 0)),
            pl.BlockSpec((B, U), lambda t: (0, 0)),
        ],
        out_shape=[
            jax.ShapeDtypeStruct((S, B, U), jnp.float32),
            jax.ShapeDtypeStruct((B, U), jnp.float32),
            jax.ShapeDtypeStruct((B, U), jnp.float32),
        ],
        scratch_shapes=[
            pltpu.VMEM((B, U), jnp.float32),
            pltpu.VMEM((B, U), jnp.float32),
        ],
        compiler_params=pltpu.CompilerParams(
            dimension_semantics=("arbitrary",)),
    )(xs2, msel, h0, c0, W, Ur, b2)


def kernel(sequence, state_h, state_c, emb_table, W, Ur, b):
    B, S = sequence.shape
    _, E = emb_table.shape
    U = state_h.shape[1]
    N = B * S
    info = plsc.get_sparse_core_info()
    NW = info.num_cores * info.num_subcores

    seqT = sequence.T.astype(jnp.int32)
    pair = (seqT >> 1).reshape(N // 128, 128)
    msel = (seqT & 1).astype(jnp.float32).reshape(S, B, 1)
    nchunks = pair.shape[0]
    full, extra = nchunks // NW, nchunks % NW
    kmax = full + (1 if extra else 0)
    starts = jnp.arange(NW) * full + jnp.minimum(jnp.arange(NW), extra)
    row_ids = jnp.minimum(starts[:, None] + jnp.arange(kmax)[None, :],
                          nchunks - 1)
    idx_pad = pair[row_ids]

    table2 = emb_table.reshape(-1, 2 * E)
    xs2 = _sc_gather(table2, idx_pad, nchunks, N).reshape(S, B, 2 * E)
    hs, hf, cf = _lstm(xs2, msel, state_h, state_c, W, Ur, b.reshape(1, -1))
    return (jnp.swapaxes(hs, 0, 1), hf, cf)


# speedup vs baseline: 4.3577x; 1.8984x over previous
import functools

import jax
import jax.numpy as jnp
from jax import lax
from jax.experimental import pallas as pl
from jax.experimental.pallas import tpu as pltpu
from jax.experimental.pallas import tpu_sc as plsc


def _sc_relayout(tableT, tail2):
    E, V = tableT.shape
    CH = 128
    n_full = V // CH
    tail = V - n_full * CH
    info = plsc.get_sparse_core_info()
    NC, NS = info.num_cores, info.num_subcores
    NW = NC * NS
    per = n_full // NW
    extra = n_full - per * NW
    mesh = plsc.VectorSubcoreMesh(core_axis_name="c", subcore_axis_name="s")

    @functools.partial(
        pl.kernel,
        mesh=mesh,
        out_type=jax.ShapeDtypeStruct((V // 2, 2 * E), jnp.float32),
        scratch_types=[
            pltpu.VMEM((2, E, CH), jnp.float32),
            pltpu.VMEM((2, CH // 2, 2 * E), jnp.float32),
            pltpu.SemaphoreType.DMA,
            pltpu.SemaphoreType.DMA,
        ],
        compiler_params=pltpu.CompilerParams(needs_layout_passes=False),
    )
    def relayout_kernel(tableT_hbm, tail2_hbm, out_hbm,
                        inbuf, outbuf, sem_i, sem_o):
        wid = lax.axis_index("s") * NC + lax.axis_index("c")
        base = wid * per + jnp.minimum(wid, extra)
        n_my = per + jnp.where(wid < extra, 1, 0)
        lanes = lax.broadcasted_iota(jnp.int32, (16,), 0)

        def col0_of(g):
            return pl.multiple_of(g * CH, CH)

        def row0_of(g):
            return pl.multiple_of(g * (CH // 2), CH // 2)

        def start_in(g, s):
            return pltpu.make_async_copy(
                tableT_hbm.at[:, pl.ds(col0_of(g), CH)],
                inbuf.at[s], sem_i)

        def transpose_block(s):
            for p in range(CH // 2):
                for h in range(2):
                    col = jnp.full((16,), 2 * p + h, jnp.int32)
                    for k in range(E // 16):
                        vals = plsc.load_gather(
                            inbuf.at[s], [k * 16 + lanes, col])
                        outbuf.at[s][p, pl.ds(h * E + k * 16, 16)] = vals

        def start_out(g, s):
            return pltpu.make_async_copy(
                outbuf.at[s],
                out_hbm.at[pl.ds(row0_of(g), CH // 2)], sem_o)

        n_tot = n_my

        def gid(c):
            return base + c

        start_in(gid(0), 0).start()

        def body(c, _):
            s = jax.lax.rem(c, 2)

            @pl.when(c + 1 < n_tot)
            def _prefetch():
                start_in(gid(c + 1), 1 - s).start()

            start_in(gid(c), s).wait()

            @pl.when(c >= 2)
            def _drain_out():
                start_out(gid(c - 2), s).wait()

            @pl.when(s == 0)
            def _t0():
                transpose_block(0)

            @pl.when(s == 1)
            def _t1():
                transpose_block(1)

            start_out(gid(c), s).start()
            return 0

        lax.fori_loop(0, n_tot, body, 0)
        start_out(0, 0).wait()
        start_out(0, 1).wait()

        if tail:
            @pl.when(wid == 5)
            def _tail_block():
                nt = tail // 2
                pltpu.async_copy(
                    tail2_hbm, outbuf.at[0, pl.ds(0, nt)], sem_i).wait()
                pltpu.async_copy(
                    outbuf.at[0, pl.ds(0, nt)],
                    out_hbm.at[pl.ds(V // 2 - nt, nt)], sem_o).wait()

    return relayout_kernel(tableT, tail2)


def _tc_pair_pack(tableT, tail2, H, BK):
    E, V = tableT.shape
    nb = H // BK
    grid = nb + 1
    T2 = tail2.shape[0]
    n_out = H + T2

    def body(a_ref, b_ref, t_ref, out_ref):
        k = pl.program_id(0)

        @pl.when(k < nb)
        def _main():
            out_ref[:, :E] = a_ref[...].T
            out_ref[:, E:] = b_ref[...].T

        @pl.when(k == nb)
        def _tail():
            out_ref[0:T2, :] = t_ref[...]

    return pl.pallas_call(
        body,
        grid=(grid,),
        in_specs=[
            pl.BlockSpec((E, BK), lambda k: (0, k)),
            pl.BlockSpec((E, BK), lambda k: (0, k + nb)),
            pl.BlockSpec((T2, 2 * E), lambda k: (0, 0)),
        ],
        out_specs=pl.BlockSpec((BK, 2 * E), lambda k: (k, 0)),
        out_shape=jax.ShapeDtypeStruct((n_out, 2 * E), jnp.float32),
        compiler_params=pltpu.CompilerParams(
            dimension_semantics=("arbitrary",)),
    )(tableT, tableT, tail2)


def _sc_gather(table, idx_pad, nchunks, n_rows):
    NW, KMAX, CH = idx_pad.shape
    _, D = table.shape
    info = plsc.get_sparse_core_info()
    NC = info.num_cores
    full = nchunks // NW
    extra = nchunks - full * NW
    mesh = plsc.VectorSubcoreMesh(core_axis_name="c", subcore_axis_name="s")

    @functools.partial(
        pl.kernel,
        mesh=mesh,
        out_type=jax.ShapeDtypeStruct((n_rows, D), table.dtype),
        scratch_types=[
            pltpu.VMEM((KMAX, CH), jnp.int32),
            pltpu.VMEM((2, CH, D), table.dtype),
            pltpu.VMEM((CH, D), table.dtype),
            pltpu.SemaphoreType.DMA,
            pltpu.SemaphoreType.DMA,
            pltpu.SemaphoreType.DMA,
        ],
    )
    def gather_kernel(table_hbm, idx_hbm, out_hbm,
                      idx_v, rows_v, rows_x, sem, semo, semx):
        wid = lax.axis_index("s") * NC + lax.axis_index("c")
        start = wid * full + jnp.minimum(wid, extra)
        pltpu.sync_copy(idx_hbm.at[wid], idx_v)

        def gather_in(j):
            return pltpu.make_async_copy(
                table_hbm.at[idx_v.at[j]], rows_v.at[j % 2], sem)

        def copy_out(j):
            return pltpu.make_async_copy(
                rows_v.at[j % 2],
                out_hbm.at[pl.ds((start + j) * CH, CH)], semo)

        if extra:
            @pl.when(wid < extra)
            def _extra_chunk():
                pltpu.async_copy(
                    table_hbm.at[idx_v.at[full]], rows_x, semx).wait()
                pltpu.sync_copy(
                    rows_x, out_hbm.at[pl.ds((start + full) * CH, CH)])

        gather_in(0).start()
        outs = []
        for j in range(full):
            gather_in(j).wait()
            if j >= 1:
                outs[j - 1].wait()
            if j + 1 < full:
                gather_in(j + 1).start()
            o = copy_out(j)
            o.start()
            outs.append(o)
        if full:
            outs[-1].wait()

    return gather_kernel(table, idx_pad)


def _lstm_body(U, E, xs_ref, m_ref, h0_ref, c0_ref, W_ref, Ur_ref, b_ref,
               out_ref, hf_ref, cf_ref, h_scr, c_scr):
    t = pl.program_id(0)
    S = pl.num_programs(0)

    @pl.when(t == 0)
    def _init():
        h_scr[...] = h0_ref[...]
        c_scr[...] = c0_ref[...]

    x2 = xs_ref[0]
    m = m_ref[0]
    xlo = x2[:, :E]
    x = xlo + m * (x2[:, E:] - xlo)
    h = h_scr[...]
    z = (jnp.dot(x, W_ref[...], preferred_element_type=jnp.float32)
         + jnp.dot(h, Ur_ref[...], preferred_element_type=jnp.float32)
         + b_ref[...])
    i = jax.nn.sigmoid(z[:, :U])
    f = jax.nn.sigmoid(z[:, U:2 * U])
    g = jnp.tanh(z[:, 2 * U:3 * U])
    o = jax.nn.sigmoid(z[:, 3 * U:])
    c_new = f * c_scr[...] + i * g
    h_new = o * jnp.tanh(c_new)
    h_scr[...] = h_new
    c_scr[...] = c_new
    out_ref[0] = h_new

    @pl.when(t == S - 1)
    def _fin():
        hf_ref[...] = h_new
        cf_ref[...] = c_new


def _lstm(xs2, msel, h0, c0, W, Ur, b2):
    S, B, E2 = xs2.shape
    E = E2 // 2
    U = h0.shape[1]
    G = 4 * U
    return pl.pallas_call(
        functools.partial(_lstm_body, U, E),
        grid=(S,),
        in_specs=[
            pl.BlockSpec((1, B, E2), lambda t: (t, 0, 0)),
            pl.BlockSpec((1, B, 1), lambda t: (t, 0, 0)),
            pl.BlockSpec((B, U), lambda t: (0, 0)),
            pl.BlockSpec((B, U), lambda t: (0, 0)),
            pl.BlockSpec((E, G), lambda t: (0, 0)),
            pl.BlockSpec((U, G), lambda t: (0, 0)),
            pl.BlockSpec((1, G), lambda t: (0, 0)),
        ],
        out_specs=[
            pl.BlockSpec((1, B, U), lambda t: (t, 0, 0)),
            pl.BlockSpec((B, U), lambda t: (0, 0)),
            pl.BlockSpec((B, U), lambda t: (0, 0)),
        ],
        out_shape=[
            jax.ShapeDtypeStruct((S, B, U), jnp.float32),
            jax.ShapeDtypeStruct((B, U), jnp.float32),
            jax.ShapeDtypeStruct((B, U), jnp.float32),
        ],
        scratch_shapes=[
            pltpu.VMEM((B, U), jnp.float32),
            pltpu.VMEM((B, U), jnp.float32),
        ],
        compiler_params=pltpu.CompilerParams(
            dimension_semantics=("arbitrary",)),
    )(xs2, msel, h0, c0, W, Ur, b2)


def kernel(sequence, state_h, state_c, emb_table, W, Ur, b):
    B, S = sequence.shape
    _, E = emb_table.shape
    U = state_h.shape[1]
    N = B * S
    info = plsc.get_sparse_core_info()
    NW = info.num_cores * info.num_subcores

    V = emb_table.shape[0]
    BKp = 3968
    nb = V // (2 * BKp)
    H = nb * BKp
    seqT = sequence.T.astype(jnp.int32)
    pair = jnp.where(seqT < 2 * H, seqT % H,
                     H + (seqT - 2 * H) // 2).reshape(N // 128, 128)
    msel = jnp.where(seqT < 2 * H, (seqT >= H).astype(jnp.int32),
                     seqT & 1).astype(jnp.float32).reshape(S, B, 1)
    nchunks = pair.shape[0]
    full, extra = nchunks // NW, nchunks % NW
    kmax = full + (1 if extra else 0)
    starts = jnp.arange(NW) * full + jnp.minimum(jnp.arange(NW), extra)
    row_ids = jnp.minimum(starts[:, None] + jnp.arange(kmax)[None, :],
                          nchunks - 1)
    idx_pad = pair[row_ids]

    tail2 = emb_table[2 * H:].reshape((V - 2 * H) // 2, 2 * E)
    table2 = _tc_pair_pack(emb_table.T, tail2, H, BKp)
    xs2 = _sc_gather(table2, idx_pad, nchunks, N).reshape(S, B, 2 * E)
    hs, hf, cf = _lstm(xs2, msel, state_h, state_c, W, Ur, b.reshape(1, -1))
    return (jnp.swapaxes(hs, 0, 1), hf, cf)


# speedup vs baseline: 4.3614x; 1.0008x over previous
import functools

import jax
import jax.numpy as jnp
from jax import lax
from jax.experimental import pallas as pl
from jax.experimental.pallas import tpu as pltpu
from jax.experimental.pallas import tpu_sc as plsc


def _tc_pair_pack(tableT, tail2, H, BK):
    E, V = tableT.shape
    nb = H // BK
    grid = nb + 1
    T2 = tail2.shape[0]
    n_out = H + T2

    def body(a_ref, b_ref, t_ref, out_ref):
        k = pl.program_id(0)

        @pl.when(k < nb)
        def _main():
            out_ref[:, :E] = a_ref[...].T
            out_ref[:, E:] = b_ref[...].T

        @pl.when(k == nb)
        def _tail():
            out_ref[0:T2, :] = t_ref[...]

    return pl.pallas_call(
        body,
        grid=(grid,),
        in_specs=[
            pl.BlockSpec((E, BK), lambda k: (0, k)),
            pl.BlockSpec((E, BK), lambda k: (0, k + nb)),
            pl.BlockSpec((T2, 2 * E), lambda k: (0, 0)),
        ],
        out_specs=pl.BlockSpec((BK, 2 * E), lambda k: (k, 0)),
        out_shape=jax.ShapeDtypeStruct((n_out, 2 * E), jnp.float32),
        compiler_params=pltpu.CompilerParams(
            dimension_semantics=("arbitrary",)),
    )(tableT, tableT, tail2)


def _sc_gather(table, idx_pad, nchunks, n_rows):
    NW, KMAX, CH = idx_pad.shape
    _, D = table.shape
    info = plsc.get_sparse_core_info()
    NC = info.num_cores
    full = nchunks // NW
    extra = nchunks - full * NW
    mesh = plsc.VectorSubcoreMesh(core_axis_name="c", subcore_axis_name="s")

    @functools.partial(
        pl.kernel,
        mesh=mesh,
        out_type=jax.ShapeDtypeStruct((n_rows, D), table.dtype),
        scratch_types=[
            pltpu.VMEM((KMAX, CH), jnp.int32),
            pltpu.VMEM((2, CH, D), table.dtype),
            pltpu.VMEM((CH, D), table.dtype),
            pltpu.SemaphoreType.DMA,
            pltpu.SemaphoreType.DMA,
            pltpu.SemaphoreType.DMA,
        ],
    )
    def gather_kernel(table_hbm, idx_hbm, out_hbm,
                      idx_v, rows_v, rows_x, sem, semo, semx):
        wid = lax.axis_index("s") * NC + lax.axis_index("c")
        start = wid * full + jnp.minimum(wid, extra)
        pltpu.sync_copy(idx_hbm.at[wid], idx_v)

        def gather_in(j):
            return pltpu.make_async_copy(
                table_hbm.at[idx_v.at[j]], rows_v.at[j % 2], sem)

        def copy_out(j):
            return pltpu.make_async_copy(
                rows_v.at[j % 2],
                out_hbm.at[pl.ds((start + j) * CH, CH)], semo)

        if extra:
            @pl.when(wid < extra)
            def _extra_chunk():
                pltpu.async_copy(
                    table_hbm.at[idx_v.at[full]], rows_x, semx).wait()
                pltpu.sync_copy(
                    rows_x, out_hbm.at[pl.ds((start + full) * CH, CH)])

        gather_in(0).start()
        outs = []
        for j in range(full):
            gather_in(j).wait()
            if j >= 1:
                outs[j - 1].wait()
            if j + 1 < full:
                gather_in(j + 1).start()
            o = copy_out(j)
            o.start()
            outs.append(o)
        if full:
            outs[-1].wait()

    return gather_kernel(table, idx_pad)


def _lstm_body(U, E, xs_ref, m_ref, h0_ref, c0_ref, W_ref, Ur_ref, b_ref,
               out_ref, hf_ref, cf_ref, h_scr, c_scr):
    t = pl.program_id(0)
    S = pl.num_programs(0)

    @pl.when(t == 0)
    def _init():
        h_scr[...] = h0_ref[...]
        c_scr[...] = c0_ref[...]

    x2 = xs_ref[0]
    m = m_ref[0]
    xlo = x2[:, :E]
    x = xlo + m * (x2[:, E:] - xlo)
    h = h_scr[...]
    z = (jnp.dot(x, W_ref[...], preferred_element_type=jnp.float32)
         + jnp.dot(h, Ur_ref[...], preferred_element_type=jnp.float32)
         + b_ref[...])
    i = jax.nn.sigmoid(z[:, :U])
    f = jax.nn.sigmoid(z[:, U:2 * U])
    g = jnp.tanh(z[:, 2 * U:3 * U])
    o = jax.nn.sigmoid(z[:, 3 * U:])
    c_new = f * c_scr[...] + i * g
    h_new = o * jnp.tanh(c_new)
    h_scr[...] = h_new
    c_scr[...] = c_new
    out_ref[0] = h_new

    @pl.when(t == S - 1)
    def _fin():
        hf_ref[...] = h_new
        cf_ref[...] = c_new


def _lstm(xs2, msel, h0, c0, W, Ur, b2):
    S, B, E2 = xs2.shape
    E = E2 // 2
    U = h0.shape[1]
    G = 4 * U
    return pl.pallas_call(
        functools.partial(_lstm_body, U, E),
        grid=(S,),
        in_specs=[
            pl.BlockSpec((1, B, E2), lambda t: (t, 0, 0)),
            pl.BlockSpec((1, B, 1), lambda t: (t, 0, 0)),
            pl.BlockSpec((B, U), lambda t: (0, 0)),
            pl.BlockSpec((B, U), lambda t: (0, 0)),
            pl.BlockSpec((E, G), lambda t: (0, 0)),
            pl.BlockSpec((U, G), lambda t: (0, 0)),
            pl.BlockSpec((1, G), lambda t: (0, 0)),
        ],
        out_specs=[
            pl.BlockSpec((1, B, U), lambda t: (t, 0, 0)),
            pl.BlockSpec((B, U), lambda t: (0, 0)),
            pl.BlockSpec((B, U), lambda t: (0, 0)),
        ],
        out_shape=[
            jax.ShapeDtypeStruct((S, B, U), jnp.float32),
            jax.ShapeDtypeStruct((B, U), jnp.float32),
            jax.ShapeDtypeStruct((B, U), jnp.float32),
        ],
        scratch_shapes=[
            pltpu.VMEM((B, U), jnp.float32),
            pltpu.VMEM((B, U), jnp.float32),
        ],
        compiler_params=pltpu.CompilerParams(
            dimension_semantics=("arbitrary",)),
    )(xs2, msel, h0, c0, W, Ur, b2)


def kernel(sequence, state_h, state_c, emb_table, W, Ur, b):
    B, S = sequence.shape
    _, E = emb_table.shape
    U = state_h.shape[1]
    N = B * S
    info = plsc.get_sparse_core_info()
    NW = info.num_cores * info.num_subcores

    V = emb_table.shape[0]
    BKp = 3968
    nb = V // (2 * BKp)
    H = nb * BKp
    seqT = sequence.T.astype(jnp.int32)
    pair = jnp.where(seqT < 2 * H, seqT % H,
                     H + (seqT - 2 * H) // 2).reshape(N // 128, 128)
    msel = jnp.where(seqT < 2 * H, (seqT >= H).astype(jnp.int32),
                     seqT & 1).astype(jnp.float32).reshape(S, B, 1)
    nchunks = pair.shape[0]
    full, extra = nchunks // NW, nchunks % NW
    kmax = full + (1 if extra else 0)
    starts = jnp.arange(NW) * full + jnp.minimum(jnp.arange(NW), extra)
    row_ids = jnp.minimum(starts[:, None] + jnp.arange(kmax)[None, :],
                          nchunks - 1)
    idx_pad = pair[row_ids]

    tail2 = emb_table[2 * H:].reshape((V - 2 * H) // 2, 2 * E)
    table2 = _tc_pair_pack(emb_table.T, tail2, H, BKp)
    xs2 = _sc_gather(table2, idx_pad, nchunks, N).reshape(S, B, 2 * E)
    hs, hf, cf = _lstm(xs2, msel, state_h, state_c, W, Ur, b.reshape(1, -1))
    return (jnp.swapaxes(hs, 0, 1), hf, cf)


# speedup vs baseline: 4.7989x; 1.1003x over previous
import functools

import jax
import jax.numpy as jnp
from jax import lax
from jax.experimental import pallas as pl
from jax.experimental.pallas import tpu as pltpu
from jax.experimental.pallas import tpu_sc as plsc


def _tc_pair_pack(tableT, tail2, H, BK):
    E, V = tableT.shape
    nb = H // BK
    grid = nb + 1
    T2 = tail2.shape[0]
    n_out = H + T2

    def body(a_ref, b_ref, t_ref, out_ref):
        k = pl.program_id(0)

        @pl.when(k < nb)
        def _main():
            out_ref[:, :E] = a_ref[...].T
            out_ref[:, E:] = b_ref[...].T

        @pl.when(k == nb)
        def _tail():
            out_ref[0:T2, :] = t_ref[...]

    return pl.pallas_call(
        body,
        grid=(grid,),
        in_specs=[
            pl.BlockSpec((E, BK), lambda k: (0, k)),
            pl.BlockSpec((E, BK), lambda k: (0, k + nb)),
            pl.BlockSpec((T2, 2 * E), lambda k: (0, 0)),
        ],
        out_specs=pl.BlockSpec((BK, 2 * E), lambda k: (k, 0)),
        out_shape=jax.ShapeDtypeStruct((n_out, 2 * E), jnp.float32),
        compiler_params=pltpu.CompilerParams(
            dimension_semantics=("arbitrary",)),
    )(tableT, tableT, tail2)


def _sc_gather(table, idx_pad, nchunks, n_rows):
    NW, KMAX, CH = idx_pad.shape
    _, D = table.shape
    info = plsc.get_sparse_core_info()
    NC = info.num_cores
    full = nchunks // NW
    extra = nchunks - full * NW
    mesh = plsc.VectorSubcoreMesh(core_axis_name="c", subcore_axis_name="s")

    @functools.partial(
        pl.kernel,
        mesh=mesh,
        out_type=jax.ShapeDtypeStruct((n_rows, D), table.dtype),
        scratch_types=[
            pltpu.VMEM((KMAX, CH), jnp.int32),
            pltpu.VMEM((2, CH, D), table.dtype),
            pltpu.VMEM((CH, D), table.dtype),
            pltpu.SemaphoreType.DMA,
            pltpu.SemaphoreType.DMA,
            pltpu.SemaphoreType.DMA,
        ],
    )
    def gather_kernel(table_hbm, idx_hbm, out_hbm,
                      idx_v, rows_v, rows_x, sem, semo, semx):
        wid = lax.axis_index("s") * NC + lax.axis_index("c")
        start = wid * full + jnp.minimum(wid, extra)
        pltpu.sync_copy(idx_hbm.at[wid], idx_v)

        def gather_in(j):
            return pltpu.make_async_copy(
                table_hbm.at[idx_v.at[j]], rows_v.at[j % 2], sem)

        def copy_out(j):
            return pltpu.make_async_copy(
                rows_v.at[j % 2],
                out_hbm.at[pl.ds((start + j) * CH, CH)], semo)

        if extra:
            @pl.when(wid < extra)
            def _extra_chunk():
                pltpu.async_copy(
                    table_hbm.at[idx_v.at[full]], rows_x, semx).wait()
                pltpu.sync_copy(
                    rows_x, out_hbm.at[pl.ds((start + full) * CH, CH)])

        gather_in(0).start()
        outs = []
        for j in range(full):
            gather_in(j).wait()
            if j >= 1:
                outs[j - 1].wait()
            if j + 1 < full:
                gather_in(j + 1).start()
            o = copy_out(j)
            o.start()
            outs.append(o)
        if full:
            outs[-1].wait()

    return gather_kernel(table, idx_pad)


def _lstm_body(U, E, xs_ref, m_ref, h0_ref, c0_ref, W_ref, Ur_ref, b_ref,
               out_ref, hf_ref, cf_ref, h_scr, c_scr):
    t = pl.program_id(0)
    S = pl.num_programs(0)

    @pl.when(t == 0)
    def _init():
        h_scr[...] = h0_ref[...]
        c_scr[...] = c0_ref[...]

    x2 = xs_ref[0]
    m = m_ref[0]
    xlo = x2[:, :E]
    x = xlo + m * (x2[:, E:] - xlo)
    h = h_scr[...]
    z = (jnp.dot(x, W_ref[...], preferred_element_type=jnp.float32)
         + jnp.dot(h, Ur_ref[...], preferred_element_type=jnp.float32)
         + b_ref[...])
    i = jax.nn.sigmoid(z[:, :U])
    f = jax.nn.sigmoid(z[:, U:2 * U])
    g = jnp.tanh(z[:, 2 * U:3 * U])
    o = jax.nn.sigmoid(z[:, 3 * U:])
    c_new = f * c_scr[...] + i * g
    h_new = o * jnp.tanh(c_new)
    h_scr[...] = h_new
    c_scr[...] = c_new
    out_ref[0] = h_new

    @pl.when(t == S - 1)
    def _fin():
        hf_ref[...] = h_new
        cf_ref[...] = c_new


def _lstm(xs2, msel, h0, c0, W, Ur, b2):
    S, B, E2 = xs2.shape
    E = E2 // 2
    U = h0.shape[1]
    G = 4 * U
    return pl.pallas_call(
        functools.partial(_lstm_body, U, E),
        grid=(S,),
        in_specs=[
            pl.BlockSpec((1, B, E2), lambda t: (t, 0, 0)),
            pl.BlockSpec((1, B, 1), lambda t: (t, 0, 0)),
            pl.BlockSpec((B, U), lambda t: (0, 0)),
            pl.BlockSpec((B, U), lambda t: (0, 0)),
            pl.BlockSpec((E, G), lambda t: (0, 0)),
            pl.BlockSpec((U, G), lambda t: (0, 0)),
            pl.BlockSpec((1, G), lambda t: (0, 0)),
        ],
        out_specs=[
            pl.BlockSpec((1, B, U), lambda t: (t, 0, 0)),
            pl.BlockSpec((B, U), lambda t: (0, 0)),
            pl.BlockSpec((B, U), lambda t: (0, 0)),
        ],
        out_shape=[
            jax.ShapeDtypeStruct((S, B, U), jnp.float32),
            jax.ShapeDtypeStruct((B, U), jnp.float32),
            jax.ShapeDtypeStruct((B, U), jnp.float32),
        ],
        scratch_shapes=[
            pltpu.VMEM((B, U), jnp.float32),
            pltpu.VMEM((B, U), jnp.float32),
        ],
        compiler_params=pltpu.CompilerParams(
            dimension_semantics=("arbitrary",)),
    )(xs2, msel, h0, c0, W, Ur, b2)


def kernel(sequence, state_h, state_c, emb_table, W, Ur, b):
    B, S = sequence.shape
    _, E = emb_table.shape
    U = state_h.shape[1]
    N = B * S
    info = plsc.get_sparse_core_info()
    NW = info.num_cores * info.num_subcores

    V = emb_table.shape[0]
    BKp = 7936
    nb = V // (2 * BKp)
    H = nb * BKp
    seqT = sequence.T.astype(jnp.int32)
    pair = jnp.where(seqT < 2 * H, seqT % H,
                     H + (seqT - 2 * H) // 2).reshape(N // 128, 128)
    msel = jnp.where(seqT < 2 * H, (seqT >= H).astype(jnp.int32),
                     seqT & 1).astype(jnp.float32).reshape(S, B, 1)
    nchunks = pair.shape[0]
    full, extra = nchunks // NW, nchunks % NW
    kmax = full + (1 if extra else 0)
    starts = jnp.arange(NW) * full + jnp.minimum(jnp.arange(NW), extra)
    row_ids = jnp.minimum(starts[:, None] + jnp.arange(kmax)[None, :],
                          nchunks - 1)
    idx_pad = pair[row_ids]

    tail2 = emb_table[2 * H:].reshape((V - 2 * H) // 2, 2 * E)
    table2 = _tc_pair_pack(emb_table.T, tail2, H, BKp)
    xs2 = _sc_gather(table2, idx_pad, nchunks, N).reshape(S, B, 2 * E)
    hs, hf, cf = _lstm(xs2, msel, state_h, state_c, W, Ur, b.reshape(1, -1))
    return (jnp.swapaxes(hs, 0, 1), hf, cf)


# speedup vs baseline: 4.8541x; 1.0115x over previous
import functools

import jax
import jax.numpy as jnp
from jax import lax
from jax.experimental import pallas as pl
from jax.experimental.pallas import tpu as pltpu
from jax.experimental.pallas import tpu_sc as plsc


def _tc_pair_pack(tableT, tail2, H, BK):
    E, V = tableT.shape
    nb = H // BK
    grid = nb + 1
    T2 = tail2.shape[0]
    n_out = H + T2

    def body(a_ref, b_ref, t_ref, out_ref):
        k = pl.program_id(0)

        @pl.when(k < nb)
        def _main():
            out_ref[:, :E] = a_ref[...].T
            out_ref[:, E:] = b_ref[...].T

        @pl.when(k == nb)
        def _tail():
            out_ref[0:T2, :] = t_ref[...]

    return pl.pallas_call(
        body,
        grid=(grid,),
        in_specs=[
            pl.BlockSpec((E, BK), lambda k: (0, k)),
            pl.BlockSpec((E, BK), lambda k: (0, k + nb)),
            pl.BlockSpec((T2, 2 * E), lambda k: (0, 0)),
        ],
        out_specs=pl.BlockSpec((BK, 2 * E), lambda k: (k, 0)),
        out_shape=jax.ShapeDtypeStruct((n_out, 2 * E), jnp.float32),
        compiler_params=pltpu.CompilerParams(
            dimension_semantics=("arbitrary",)),
    )(tableT, tableT, tail2)


def _sc_gather(table, idx_pad, nchunks, n_rows):
    NW, KMAX, CH = idx_pad.shape
    _, D = table.shape
    info = plsc.get_sparse_core_info()
    NC = info.num_cores
    full = nchunks // NW
    extra = nchunks - full * NW
    mesh = plsc.VectorSubcoreMesh(core_axis_name="c", subcore_axis_name="s")

    @functools.partial(
        pl.kernel,
        mesh=mesh,
        out_type=jax.ShapeDtypeStruct((n_rows, D), table.dtype),
        scratch_types=[
            pltpu.VMEM((KMAX, CH), jnp.int32),
            pltpu.VMEM((2, CH, D), table.dtype),
            pltpu.VMEM((CH, D), table.dtype),
            pltpu.SemaphoreType.DMA,
            pltpu.SemaphoreType.DMA,
            pltpu.SemaphoreType.DMA,
        ],
    )
    def gather_kernel(table_hbm, idx_hbm, out_hbm,
                      idx_v, rows_v, rows_x, sem, semo, semx):
        wid = lax.axis_index("s") * NC + lax.axis_index("c")
        start = wid * full + jnp.minimum(wid, extra)
        pltpu.sync_copy(idx_hbm.at[wid], idx_v)

        def gather_in(j):
            return pltpu.make_async_copy(
                table_hbm.at[idx_v.at[j]], rows_v.at[j % 2], sem)

        def copy_out(j):
            return pltpu.make_async_copy(
                rows_v.at[j % 2],
                out_hbm.at[pl.ds((start + j) * CH, CH)], semo)

        if extra:
            @pl.when(wid < extra)
            def _extra_chunk():
                pltpu.async_copy(
                    table_hbm.at[idx_v.at[full]], rows_x, semx).wait()
                pltpu.sync_copy(
                    rows_x, out_hbm.at[pl.ds((start + full) * CH, CH)])

        gather_in(0).start()
        outs = []
        for j in range(full):
            gather_in(j).wait()
            if j >= 1:
                outs[j - 1].wait()
            if j + 1 < full:
                gather_in(j + 1).start()
            o = copy_out(j)
            o.start()
            outs.append(o)
        if full:
            outs[-1].wait()

    return gather_kernel(table, idx_pad)


def _lstm_body(U, E, xs_ref, m_ref, h0_ref, c0_ref, W_ref, Ur_ref, b_ref,
               out_ref, hf_ref, cf_ref, h_scr, c_scr):
    t = pl.program_id(0)
    S = pl.num_programs(0)

    @pl.when(t == 0)
    def _init():
        h_scr[...] = h0_ref[...]
        c_scr[...] = c0_ref[...]

    x2 = xs_ref[0]
    m = m_ref[0]
    xlo = x2[:, :E]
    x = xlo + m * (x2[:, E:] - xlo)
    h = h_scr[...]
    z = (jnp.dot(x, W_ref[...], preferred_element_type=jnp.float32)
         + jnp.dot(h, Ur_ref[...], preferred_element_type=jnp.float32)
         + b_ref[...])
    i = jax.nn.sigmoid(z[:, :U])
    f = jax.nn.sigmoid(z[:, U:2 * U])
    g = jnp.tanh(z[:, 2 * U:3 * U])
    o = jax.nn.sigmoid(z[:, 3 * U:])
    c_new = f * c_scr[...] + i * g
    h_new = o * jnp.tanh(c_new)
    h_scr[...] = h_new
    c_scr[...] = c_new
    out_ref[0] = h_new

    @pl.when(t == S - 1)
    def _fin():
        hf_ref[...] = h_new
        cf_ref[...] = c_new


def _lstm(xs2, msel, h0, c0, W, Ur, b2):
    S, B, E2 = xs2.shape
    E = E2 // 2
    U = h0.shape[1]
    G = 4 * U
    return pl.pallas_call(
        functools.partial(_lstm_body, U, E),
        grid=(S,),
        in_specs=[
            pl.BlockSpec((1, B, E2), lambda t: (t, 0, 0)),
            pl.BlockSpec((1, B, 1), lambda t: (t, 0, 0)),
            pl.BlockSpec((B, U), lambda t: (0, 0)),
            pl.BlockSpec((B, U), lambda t: (0, 0)),
            pl.BlockSpec((E, G), lambda t: (0, 0)),
            pl.BlockSpec((U, G), lambda t: (0, 0)),
            pl.BlockSpec((1, G), lambda t: (0, 0)),
        ],
        out_specs=[
            pl.BlockSpec((1, B, U), lambda t: (t, 0, 0)),
            pl.BlockSpec((B, U), lambda t: (0, 0)),
            pl.BlockSpec((B, U), lambda t: (0, 0)),
        ],
        out_shape=[
            jax.ShapeDtypeStruct((S, B, U), jnp.float32),
            jax.ShapeDtypeStruct((B, U), jnp.float32),
            jax.ShapeDtypeStruct((B, U), jnp.float32),
        ],
        scratch_shapes=[
            pltpu.VMEM((B, U), jnp.float32),
            pltpu.VMEM((B, U), jnp.float32),
        ],
        compiler_params=pltpu.CompilerParams(
            dimension_semantics=("arbitrary",)),
    )(xs2, msel, h0, c0, W, Ur, b2)


def kernel(sequence, state_h, state_c, emb_table, W, Ur, b):
    B, S = sequence.shape
    _, E = emb_table.shape
    U = state_h.shape[1]
    N = B * S
    info = plsc.get_sparse_core_info()
    NW = info.num_cores * info.num_subcores

    V = emb_table.shape[0]
    BKp = 15872
    nb = V // (2 * BKp)
    H = nb * BKp
    seqT = sequence.T.astype(jnp.int32)
    pair = jnp.where(seqT < 2 * H, seqT % H,
                     H + (seqT - 2 * H) // 2).reshape(N // 128, 128)
    msel = jnp.where(seqT < 2 * H, (seqT >= H).astype(jnp.int32),
                     seqT & 1).astype(jnp.float32).reshape(S, B, 1)
    nchunks = pair.shape[0]
    full, extra = nchunks // NW, nchunks % NW
    kmax = full + (1 if extra else 0)
    starts = jnp.arange(NW) * full + jnp.minimum(jnp.arange(NW), extra)
    row_ids = jnp.minimum(starts[:, None] + jnp.arange(kmax)[None, :],
                          nchunks - 1)
    idx_pad = pair[row_ids]

    tail2 = emb_table[2 * H:].reshape((V - 2 * H) // 2, 2 * E)
    table2 = _tc_pair_pack(emb_table.T, tail2, H, BKp)
    xs2 = _sc_gather(table2, idx_pad, nchunks, N).reshape(S, B, 2 * E)
    hs, hf, cf = _lstm(xs2, msel, state_h, state_c, W, Ur, b.reshape(1, -1))
    return (jnp.swapaxes(hs, 0, 1), hf, cf)
